# baseline (speedup 1.0000x reference)
import functools

import jax
import jax.numpy as jnp
from jax import lax
from jax.experimental import pallas as pl
from jax.experimental.pallas import tpu as pltpu
from jax.experimental.pallas import tpu_sc as plsc

BATCH = 16384
N_FIELDS = 26
EMB = 16

NW = 32
N = BATCH * N_FIELDS
PER_W = N // NW
CHUNK = 128
CPW = PER_W // CHUNK
SUPER = 8
NSUP = CPW // SUPER


def _gather_body(idx_hbm, table_hbm, out_hbm, idx_v, rows_v, gsem, wsem):
    wid = lax.axis_index("s") * 2 + lax.axis_index("c")
    pltpu.sync_copy(idx_hbm.at[wid], idx_v)

    def superstep(s, carry):
        buf = lax.rem(s, 2)
        copies = []
        for b in range(SUPER):
            copies.append(pltpu.async_copy(
                table_hbm.at[idx_v.at[s * SUPER + b]],
                rows_v.at[buf, pl.ds(b * CHUNK, CHUNK)],
                gsem,
            ))
        for c in copies:
            c.wait()
        row0 = wid * PER_W + s * (SUPER * CHUNK)
        pltpu.async_copy(
            rows_v.at[buf],
            out_hbm.at[pl.ds(row0, SUPER * CHUNK)],
            wsem,
        ).wait()
        return carry

    lax.fori_loop(0, NSUP, superstep, 0)


@jax.jit
def _gather(table, idx3):
    mesh = plsc.VectorSubcoreMesh(core_axis_name="c", subcore_axis_name="s")
    f = pl.kernel(
        _gather_body,
        out_type=jax.ShapeDtypeStruct((N, EMB), jnp.float32),
        mesh=mesh,
        scratch_types=[
            pltpu.VMEM((CPW, CHUNK), jnp.int32),
            pltpu.VMEM((2, SUPER * CHUNK, EMB), jnp.float32),
            pltpu.SemaphoreType.DMA,
            pltpu.SemaphoreType.DMA,
        ],
        compiler_params=pltpu.CompilerParams(use_tc_tiling_on_sc=False),
    )
    return f(idx3, table)


def kernel(move_name, move_embed_weight):
    idx3 = move_name.astype(jnp.int32).reshape(NW, CPW, CHUNK)
    out = _gather(move_embed_weight, idx3)
    return out.reshape(BATCH, N_FIELDS, EMB)


# speedup vs baseline: 1.0078x; 1.0078x over previous
import functools

import jax
import jax.numpy as jnp
from jax import lax
from jax.experimental import pallas as pl
from jax.experimental.pallas import tpu as pltpu
from jax.experimental.pallas import tpu_sc as plsc

BATCH = 16384
N_FIELDS = 26
EMB = 16

NW = 32
N = BATCH * N_FIELDS
PER_W = N // NW
CHUNK = 128
CPW = PER_W // CHUNK
SUPER = 8
NSUP = CPW // SUPER


NBUF = 4


def _gather_body(idx_hbm, table_hbm, out_hbm, idx_v, rows_v, gsem, wsem):
    wid = lax.axis_index("s") * 2 + lax.axis_index("c")
    pltpu.sync_copy(idx_hbm.at[wid], idx_v)

    def fire_gathers(s, buf):
        for b in range(SUPER):
            pltpu.async_copy(
                table_hbm.at[idx_v.at[s * SUPER + b]],
                rows_v.at[buf, pl.ds(b * CHUNK, CHUNK)],
                gsem,
            )

    def wait_gathers(buf):
        for b in range(SUPER):
            pltpu.make_async_copy(
                table_hbm.at[idx_v.at[0]],
                rows_v.at[buf, pl.ds(b * CHUNK, CHUNK)],
                gsem,
            ).wait()

    def fire_write(s, buf):
        pltpu.async_copy(
            rows_v.at[buf],
            out_hbm.at[pl.ds(wid * PER_W + s * (SUPER * CHUNK), SUPER * CHUNK)],
            wsem,
        )

    def wait_write(buf):
        pltpu.make_async_copy(
            rows_v.at[buf],
            out_hbm.at[pl.ds(0, SUPER * CHUNK)],
            wsem,
        ).wait()

    fire_gathers(0, 0)
    fire_gathers(1, 1)

    def superstep(s, carry):
        buf = lax.rem(s, NBUF)
        wait_gathers(buf)
        fire_write(s, buf)

        @pl.when(s + 2 < NSUP)
        def _():
            @pl.when(s >= 2)
            def _():
                wait_write(lax.rem(s + 2, NBUF))
            fire_gathers(s + 2, lax.rem(s + 2, NBUF))
        return carry

    lax.fori_loop(0, NSUP, superstep, 0)
    for i in range(max(NSUP - NBUF, 0), NSUP):
        wait_write(lax.rem(i, NBUF))


@jax.jit
def _gather(table, idx3):
    mesh = plsc.VectorSubcoreMesh(core_axis_name="c", subcore_axis_name="s")
    f = pl.kernel(
        _gather_body,
        out_type=jax.ShapeDtypeStruct((N, EMB), jnp.float32),
        mesh=mesh,
        scratch_types=[
            pltpu.VMEM((CPW, CHUNK), jnp.int32),
            pltpu.VMEM((NBUF, SUPER * CHUNK, EMB), jnp.float32),
            pltpu.SemaphoreType.DMA,
            pltpu.SemaphoreType.DMA,
        ],
        compiler_params=pltpu.CompilerParams(use_tc_tiling_on_sc=False),
    )
    return f(idx3, table)


def kernel(move_name, move_embed_weight):
    idx3 = move_name.astype(jnp.int32).reshape(NW, CPW, CHUNK)
    out = _gather(move_embed_weight, idx3)
    return out.reshape(BATCH, N_FIELDS, EMB)


# speedup vs baseline: 1.4382x; 1.4270x over previous
import jax
import jax.numpy as jnp
from jax import lax
from jax.experimental import pallas as pl
from jax.experimental.pallas import tpu as pltpu
from jax.experimental.pallas import tpu_sc as plsc

BATCH = 16384
NF = 26
EMB = 16

CHUNK_B = 1024
NCH = BATCH // CHUNK_B
GSUB = 128
NG = CHUNK_B // GSUB
TB = CHUNK_B // 128


def _body(idxT_hbm, table_hbm, out_hbm, idx_v, rows0, rows1, gsem, wsem):
    wid = lax.axis_index("s") * 2 + lax.axis_index("c")

    def fire_gathers(c, rows_v):
        for g in range(NG):
            pltpu.async_copy(
                table_hbm.at[idx_v.at[pl.ds(c * CHUNK_B + g * GSUB, GSUB)]],
                rows_v.at[g],
                gsem,
            )

    def wait_gathers(rows_v):
        for g in range(NG):
            pltpu.make_async_copy(
                table_hbm.at[idx_v.at[pl.ds(0, GSUB)]],
                rows_v.at[g],
                gsem,
            ).wait()

    def fire_writes(c, rows_v):
        for g in range(NG):
            pltpu.async_copy(
                rows_v.at[g],
                out_hbm.at[wid, pl.ds(c * CHUNK_B + g * GSUB, GSUB)],
                wsem,
            )

    def wait_writes():
        for g in range(NG):
            pltpu.make_async_copy(
                rows0.at[0],
                out_hbm.at[0, pl.ds(0, GSUB)],
                wsem,
            ).wait()

    @pl.when(wid < NF)
    def _():
        pltpu.sync_copy(idxT_hbm.at[wid], idx_v)
        fire_gathers(0, rows0)

        def dstep(k, carry):
            c0 = 2 * k
            c1 = 2 * k + 1

            wait_gathers(rows0)
            fire_writes(c0, rows0)

            @pl.when(k >= 1)
            def _():
                wait_writes()

            fire_gathers(c1, rows1)

            wait_gathers(rows1)
            fire_writes(c1, rows1)
            wait_writes()

            @pl.when(c1 + 1 < NCH)
            def _():
                fire_gathers(c1 + 1, rows0)

            return carry

        lax.fori_loop(0, NCH // 2, dstep, 0)
        wait_writes()


def _build():
    mesh = plsc.VectorSubcoreMesh(core_axis_name="c", subcore_axis_name="s")
    return pl.kernel(
        _body,
        out_type=jax.ShapeDtypeStruct((NF, BATCH, EMB), jnp.float32),
        mesh=mesh,
        scratch_types=[
            pltpu.VMEM((BATCH,), jnp.int32),
            pltpu.VMEM((TB, GSUB, EMB), jnp.float32),
            pltpu.VMEM((TB, GSUB, EMB), jnp.float32),
            pltpu.SemaphoreType.DMA,
            pltpu.SemaphoreType.DMA,
        ],
        compiler_params=pltpu.CompilerParams(use_tc_tiling_on_sc=False),
    )


def kernel(move_name, move_embed_weight):
    idxT = move_name.T
    L = _build()(idxT, move_embed_weight)
    return L.transpose((1, 0, 2))
